# initial kernel scaffold (unmeasured)
import jax
import jax.numpy as jnp
from jax import lax
from jax.experimental import pallas as pl
from jax.experimental.pallas import tpu as pltpu


def kernel(
    x,
):
    def body(*refs):
        pass

    out_shape = jax.ShapeDtypeStruct(..., jnp.float32)
    return pl.pallas_call(body, out_shape=out_shape)(...)


# baseline (device time: 153134 ns/iter reference)
import jax
import jax.numpy as jnp
from jax import lax
from jax.experimental import pallas as pl
from jax.experimental.pallas import tpu as pltpu

M = 4096
N = 4096
T = 512
G = M // T
RH = 8
CH = 128


def kernel(x):
    def body(x_hbm, o_hbm, buf, out_buf, row_halo, col_halo,
             local_sems, send_sems, recv_sems):
        i = pl.program_id(0)
        my_x = lax.axis_index("x")
        my_y = lax.axis_index("y")
        x_nbr = (1 - my_x, my_y)
        y_nbr = (my_x, 1 - my_y)

        @pl.when(i == 0)
        def _comm():
            barrier_sem = pltpu.get_barrier_semaphore()
            for nbr in (x_nbr, y_nbr):
                pl.semaphore_signal(
                    barrier_sem, inc=1,
                    device_id=nbr, device_id_type=pl.DeviceIdType.MESH,
                )
            pl.semaphore_wait(barrier_sem, 2)

            send_row = (1 - my_x) * (M - RH)
            row_rdma = pltpu.make_async_remote_copy(
                src_ref=x_hbm.at[pl.ds(send_row, RH), :],
                dst_ref=row_halo,
                send_sem=send_sems.at[0],
                recv_sem=recv_sems.at[0],
                device_id=x_nbr,
                device_id_type=pl.DeviceIdType.MESH,
            )
            row_rdma.start()

            send_col = (1 - my_y) * (N - CH)
            col_rdma = pltpu.make_async_remote_copy(
                src_ref=x_hbm.at[:, pl.ds(send_col, CH)],
                dst_ref=col_halo,
                send_sem=send_sems.at[1],
                recv_sem=recv_sems.at[1],
                device_id=y_nbr,
                device_id_type=pl.DeviceIdType.MESH,
            )
            col_rdma.start()
            row_rdma.wait()
            col_rdma.wait()

        r0 = i * T
        main_cp = pltpu.make_async_copy(
            x_hbm.at[pl.ds(r0, T), :], buf.at[pl.ds(RH, T), :], local_sems.at[0]
        )
        main_cp.start()

        @pl.when(i > 0)
        def _top_local():
            c = pltpu.make_async_copy(
                x_hbm.at[pl.ds(r0 - RH, RH), :], buf.at[pl.ds(0, RH), :],
                local_sems.at[1],
            )
            c.start()
            c.wait()

        @pl.when((i == 0) & (my_x == 1))
        def _top_halo():
            c = pltpu.make_async_copy(
                row_halo, buf.at[pl.ds(0, RH), :], local_sems.at[1]
            )
            c.start()
            c.wait()

        @pl.when(i < G - 1)
        def _bot_local():
            c = pltpu.make_async_copy(
                x_hbm.at[pl.ds(r0 + T, RH), :], buf.at[pl.ds(T + RH, RH), :],
                local_sems.at[2],
            )
            c.start()
            c.wait()

        @pl.when((i == G - 1) & (my_x == 0))
        def _bot_halo():
            c = pltpu.make_async_copy(
                row_halo, buf.at[pl.ds(T + RH, RH), :], local_sems.at[2]
            )
            c.start()
            c.wait()

        main_cp.wait()

        center = buf[RH:T + RH, :]
        north = buf[RH - 1:T + RH - 1, :]
        south = buf[RH + 1:T + RH + 1, :]
        hblk = col_halo[pl.ds(r0, T), :]
        hcol = jnp.where(my_y == 1, hblk[:, CH - 1:CH], hblk[:, 0:1])
        west = jnp.concatenate([hcol, center[:, :N - 1]], axis=1)
        east = jnp.concatenate([center[:, 1:], hcol], axis=1)

        stencil = 0.5 * center + 0.125 * (north + south + west + east)

        row_ids = r0 + lax.broadcasted_iota(jnp.int32, (T, N), 0)
        col_ids = lax.broadcasted_iota(jnp.int32, (T, N), 1)
        boundary = (
            ((my_x == 0) & (row_ids == 0))
            | ((my_x == 1) & (row_ids == M - 1))
            | ((my_y == 0) & (col_ids == 0))
            | ((my_y == 1) & (col_ids == N - 1))
        )
        out_buf[:, :] = jnp.where(boundary, center, stencil).astype(jnp.bfloat16)

        ocp = pltpu.make_async_copy(
            out_buf, o_hbm.at[pl.ds(r0, T), :], local_sems.at[0]
        )
        ocp.start()
        ocp.wait()

    return pl.pallas_call(
        body,
        grid=(G,),
        out_shape=jax.ShapeDtypeStruct((M, N), jnp.bfloat16),
        in_specs=[pl.BlockSpec(memory_space=pl.ANY)],
        out_specs=pl.BlockSpec(memory_space=pl.ANY),
        scratch_shapes=[
            pltpu.VMEM((T + 2 * RH, N), jnp.float32),
            pltpu.VMEM((T, N), jnp.bfloat16),
            pltpu.VMEM((RH, N), jnp.float32),
            pltpu.VMEM((M, CH), jnp.float32),
            pltpu.SemaphoreType.DMA((3,)),
            pltpu.SemaphoreType.DMA((2,)),
            pltpu.SemaphoreType.DMA((2,)),
        ],
        compiler_params=pltpu.CompilerParams(
            collective_id=0,
            dimension_semantics=("arbitrary",),
        ),
    )(x)


# device time: 89580 ns/iter; 1.7095x vs baseline; 1.7095x over previous
import jax
import jax.numpy as jnp
from jax import lax
from jax.experimental import pallas as pl
from jax.experimental.pallas import tpu as pltpu

M = 4096
N = 4096
T = 512
G = M // T
RH = 8
CH = 128


def kernel(x):
    def body(x_hbm, o_hbm, buf, out_buf, row_halo, col_halo,
             load_sems, out_sems, row_send, row_recv, col_send, col_recv):
        i = pl.program_id(0)
        slot = lax.rem(i, 2)
        nslot = lax.rem(i + 1, 2)
        my_x = lax.axis_index("x")
        my_y = lax.axis_index("y")
        x_nbr = (1 - my_x, my_y)
        y_nbr = (my_x, 1 - my_y)
        send_col = (1 - my_y) * (N - CH)

        def col_chunk(k):
            return pltpu.make_async_remote_copy(
                src_ref=x_hbm.at[pl.ds(k * T, T), pl.ds(send_col, CH)],
                dst_ref=col_halo.at[pl.ds(k * T, T), :],
                send_sem=col_send.at[k],
                recv_sem=col_recv.at[k],
                device_id=y_nbr,
                device_id_type=pl.DeviceIdType.MESH,
            )

        @pl.when(i == 0)
        def _prologue():
            barrier_sem = pltpu.get_barrier_semaphore()
            for nbr in (x_nbr, y_nbr):
                pl.semaphore_signal(
                    barrier_sem, inc=1,
                    device_id=nbr, device_id_type=pl.DeviceIdType.MESH,
                )
            pl.semaphore_wait(barrier_sem, 2)

            send_row = (1 - my_x) * (M - RH)
            row_rdma = pltpu.make_async_remote_copy(
                src_ref=x_hbm.at[pl.ds(send_row, RH), :],
                dst_ref=row_halo,
                send_sem=row_send,
                recv_sem=row_recv,
                device_id=x_nbr,
                device_id_type=pl.DeviceIdType.MESH,
            )
            row_rdma.start()
            for k in range(G):
                col_chunk(k).start()

            pltpu.make_async_copy(
                x_hbm.at[pl.ds(0, T), :], buf.at[0, pl.ds(RH, T), :],
                load_sems.at[0, 0],
            ).start()
            pltpu.make_async_copy(
                x_hbm.at[pl.ds(T, RH), :], buf.at[0, pl.ds(T + RH, RH), :],
                load_sems.at[0, 2],
            ).start()
            row_rdma.wait()
            pltpu.make_async_copy(
                row_halo, buf.at[0, pl.ds(0, RH), :], load_sems.at[0, 1]
            ).start()

        @pl.when(i < G - 1)
        def _prefetch():
            k0 = (i + 1) * T
            pltpu.make_async_copy(
                x_hbm.at[pl.ds(k0, T), :], buf.at[nslot, pl.ds(RH, T), :],
                load_sems.at[nslot, 0],
            ).start()
            pltpu.make_async_copy(
                x_hbm.at[pl.ds(k0 - RH, RH), :], buf.at[nslot, pl.ds(0, RH), :],
                load_sems.at[nslot, 1],
            ).start()

            @pl.when(i + 1 < G - 1)
            def _bot_local():
                pltpu.make_async_copy(
                    x_hbm.at[pl.ds(k0 + T, RH), :],
                    buf.at[nslot, pl.ds(T + RH, RH), :],
                    load_sems.at[nslot, 2],
                ).start()

            @pl.when(i + 1 == G - 1)
            def _bot_halo():
                pltpu.make_async_copy(
                    row_halo, buf.at[nslot, pl.ds(T + RH, RH), :],
                    load_sems.at[nslot, 2],
                ).start()

        pltpu.make_async_copy(
            x_hbm.at[pl.ds(i * T, T), :], buf.at[slot, pl.ds(RH, T), :],
            load_sems.at[slot, 0],
        ).wait()
        pltpu.make_async_copy(
            row_halo, buf.at[slot, pl.ds(0, RH), :], load_sems.at[slot, 1]
        ).wait()
        pltpu.make_async_copy(
            row_halo, buf.at[slot, pl.ds(T + RH, RH), :], load_sems.at[slot, 2]
        ).wait()
        col_chunk(i).wait()

        @pl.when(i >= 2)
        def _free_out():
            pltpu.make_async_copy(
                out_buf.at[slot], o_hbm.at[pl.ds((i - 2) * T, T), :],
                out_sems.at[slot],
            ).wait()

        center = buf[slot, RH:T + RH, :]
        north = buf[slot, RH - 1:T + RH - 1, :]
        south = buf[slot, RH + 1:T + RH + 1, :]
        r0 = i * T
        hblk = col_halo[pl.ds(r0, T), :]
        hcol = jnp.where(my_y == 1, hblk[:, CH - 1:CH], hblk[:, 0:1])
        west = jnp.concatenate([hcol, center[:, :N - 1]], axis=1)
        east = jnp.concatenate([center[:, 1:], hcol], axis=1)

        stencil = 0.5 * center + 0.125 * (north + south + west + east)

        row_ids = r0 + lax.broadcasted_iota(jnp.int32, (T, N), 0)
        col_ids = lax.broadcasted_iota(jnp.int32, (T, N), 1)
        boundary = (
            ((my_x == 0) & (row_ids == 0))
            | ((my_x == 1) & (row_ids == M - 1))
            | ((my_y == 0) & (col_ids == 0))
            | ((my_y == 1) & (col_ids == N - 1))
        )
        out_buf[slot, :, :] = jnp.where(
            boundary, center, stencil
        ).astype(jnp.bfloat16)

        ocp = pltpu.make_async_copy(
            out_buf.at[slot], o_hbm.at[pl.ds(r0, T), :], out_sems.at[slot]
        )
        ocp.start()

        @pl.when(i == G - 1)
        def _drain():
            pltpu.make_async_copy(
                out_buf.at[nslot], o_hbm.at[pl.ds((G - 2) * T, T), :],
                out_sems.at[nslot],
            ).wait()
            ocp.wait()

    return pl.pallas_call(
        body,
        grid=(G,),
        out_shape=jax.ShapeDtypeStruct((M, N), jnp.bfloat16),
        in_specs=[pl.BlockSpec(memory_space=pl.ANY)],
        out_specs=pl.BlockSpec(memory_space=pl.ANY),
        scratch_shapes=[
            pltpu.VMEM((2, T + 2 * RH, N), jnp.float32),
            pltpu.VMEM((2, T, N), jnp.bfloat16),
            pltpu.VMEM((RH, N), jnp.float32),
            pltpu.VMEM((M, CH), jnp.float32),
            pltpu.SemaphoreType.DMA((2, 3)),
            pltpu.SemaphoreType.DMA((2,)),
            pltpu.SemaphoreType.DMA,
            pltpu.SemaphoreType.DMA,
            pltpu.SemaphoreType.DMA((G,)),
            pltpu.SemaphoreType.DMA((G,)),
        ],
        compiler_params=pltpu.CompilerParams(
            collective_id=0,
            dimension_semantics=("arbitrary",),
            vmem_limit_bytes=64 * 1024 * 1024,
        ),
    )(x)


# device time: 83527 ns/iter; 1.8333x vs baseline; 1.0725x over previous
import jax
import jax.numpy as jnp
from jax import lax
from jax.experimental import pallas as pl
from jax.experimental.pallas import tpu as pltpu

M = 4096
N = 4096
T = 512
G = M // T
RH = 8
CH = 128


def kernel(x):
    def body(x_hbm, o_hbm, buf, out_buf, row_halo, col_halo,
             load_sems, out_sems, row_send, row_recv, col_send, col_recv):
        i = pl.program_id(0)
        slot = lax.rem(i, 2)
        nslot = lax.rem(i + 1, 2)
        my_x = lax.axis_index("x")
        my_y = lax.axis_index("y")
        x_nbr = (1 - my_x, my_y)
        y_nbr = (my_x, 1 - my_y)
        send_col = (1 - my_y) * (N - CH)

        def col_chunk(k):
            return pltpu.make_async_remote_copy(
                src_ref=x_hbm.at[pl.ds(k * T, T), pl.ds(send_col, CH)],
                dst_ref=col_halo.at[pl.ds(k * T, T), :],
                send_sem=col_send.at[k],
                recv_sem=col_recv.at[k],
                device_id=y_nbr,
                device_id_type=pl.DeviceIdType.MESH,
            )

        @pl.when(i == 0)
        def _prologue():
            pltpu.make_async_copy(
                x_hbm.at[pl.ds(0, T), :], buf.at[0, pl.ds(RH, T), :],
                load_sems.at[0, 0],
            ).start()
            pltpu.make_async_copy(
                x_hbm.at[pl.ds(T, RH), :], buf.at[0, pl.ds(T + RH, RH), :],
                load_sems.at[0, 2],
            ).start()

            barrier_sem = pltpu.get_barrier_semaphore()
            for nbr in (x_nbr, y_nbr):
                pl.semaphore_signal(
                    barrier_sem, inc=1,
                    device_id=nbr, device_id_type=pl.DeviceIdType.MESH,
                )
            pl.semaphore_wait(barrier_sem, 2)

            send_row = (1 - my_x) * (M - RH)
            row_rdma = pltpu.make_async_remote_copy(
                src_ref=x_hbm.at[pl.ds(send_row, RH), :],
                dst_ref=row_halo,
                send_sem=row_send,
                recv_sem=row_recv,
                device_id=x_nbr,
                device_id_type=pl.DeviceIdType.MESH,
            )
            row_rdma.start()
            for k in range(G):
                col_chunk(k).start()
            row_rdma.wait()
            pltpu.make_async_copy(
                row_halo, buf.at[0, pl.ds(0, RH), :], load_sems.at[0, 1]
            ).start()

        @pl.when(i < G - 1)
        def _prefetch():
            k0 = (i + 1) * T
            pltpu.make_async_copy(
                x_hbm.at[pl.ds(k0, T), :], buf.at[nslot, pl.ds(RH, T), :],
                load_sems.at[nslot, 0],
            ).start()
            pltpu.make_async_copy(
                x_hbm.at[pl.ds(k0 - RH, RH), :], buf.at[nslot, pl.ds(0, RH), :],
                load_sems.at[nslot, 1],
            ).start()

            @pl.when(i + 1 < G - 1)
            def _bot_local():
                pltpu.make_async_copy(
                    x_hbm.at[pl.ds(k0 + T, RH), :],
                    buf.at[nslot, pl.ds(T + RH, RH), :],
                    load_sems.at[nslot, 2],
                ).start()

            @pl.when(i + 1 == G - 1)
            def _bot_halo():
                pltpu.make_async_copy(
                    row_halo, buf.at[nslot, pl.ds(T + RH, RH), :],
                    load_sems.at[nslot, 2],
                ).start()

        pltpu.make_async_copy(
            x_hbm.at[pl.ds(i * T, T), :], buf.at[slot, pl.ds(RH, T), :],
            load_sems.at[slot, 0],
        ).wait()
        pltpu.make_async_copy(
            row_halo, buf.at[slot, pl.ds(0, RH), :], load_sems.at[slot, 1]
        ).wait()
        pltpu.make_async_copy(
            row_halo, buf.at[slot, pl.ds(T + RH, RH), :], load_sems.at[slot, 2]
        ).wait()
        col_chunk(i).wait()

        @pl.when(i >= 2)
        def _free_out():
            pltpu.make_async_copy(
                out_buf.at[slot], o_hbm.at[pl.ds((i - 2) * T, T), :],
                out_sems.at[slot],
            ).wait()

        center = buf[slot, RH:T + RH, :]
        north = buf[slot, RH - 1:T + RH - 1, :]
        south = buf[slot, RH + 1:T + RH + 1, :]
        r0 = i * T
        hblk = col_halo[pl.ds(r0, T), :]
        hcol = jnp.where(my_y == 1, hblk[:, CH - 1:CH], hblk[:, 0:1])
        west = jnp.concatenate([hcol, center[:, :N - 1]], axis=1)
        east = jnp.concatenate([center[:, 1:], hcol], axis=1)

        stencil = 0.5 * center + 0.125 * (north + south + west + east)
        out_buf[slot, :, :] = stencil.astype(jnp.bfloat16)

        @pl.when(my_y == 0)
        def _west_edge():
            out_buf[slot, :, 0:1] = center[:, 0:1].astype(jnp.bfloat16)

        @pl.when(my_y == 1)
        def _east_edge():
            out_buf[slot, :, N - 1:N] = center[:, N - 1:N].astype(jnp.bfloat16)

        @pl.when((i == 0) & (my_x == 0))
        def _north_edge():
            out_buf[slot, 0:1, :] = center[0:1, :].astype(jnp.bfloat16)

        @pl.when((i == G - 1) & (my_x == 1))
        def _south_edge():
            out_buf[slot, T - 1:T, :] = center[T - 1:T, :].astype(jnp.bfloat16)

        ocp = pltpu.make_async_copy(
            out_buf.at[slot], o_hbm.at[pl.ds(r0, T), :], out_sems.at[slot]
        )
        ocp.start()

        @pl.when(i == G - 1)
        def _drain():
            pltpu.make_async_copy(
                out_buf.at[nslot], o_hbm.at[pl.ds((G - 2) * T, T), :],
                out_sems.at[nslot],
            ).wait()
            ocp.wait()

    return pl.pallas_call(
        body,
        grid=(G,),
        out_shape=jax.ShapeDtypeStruct((M, N), jnp.bfloat16),
        in_specs=[pl.BlockSpec(memory_space=pl.ANY)],
        out_specs=pl.BlockSpec(memory_space=pl.ANY),
        scratch_shapes=[
            pltpu.VMEM((2, T + 2 * RH, N), jnp.float32),
            pltpu.VMEM((2, T, N), jnp.bfloat16),
            pltpu.VMEM((RH, N), jnp.float32),
            pltpu.VMEM((M, CH), jnp.float32),
            pltpu.SemaphoreType.DMA((2, 3)),
            pltpu.SemaphoreType.DMA((2,)),
            pltpu.SemaphoreType.DMA,
            pltpu.SemaphoreType.DMA,
            pltpu.SemaphoreType.DMA((G,)),
            pltpu.SemaphoreType.DMA((G,)),
        ],
        compiler_params=pltpu.CompilerParams(
            collective_id=0,
            dimension_semantics=("arbitrary",),
            vmem_limit_bytes=64 * 1024 * 1024,
        ),
    )(x)
